# baseline (speedup 1.0000x reference)
import functools

import jax
import jax.numpy as jnp
from jax import lax
from jax.experimental import pallas as pl
from jax.experimental.pallas import tpu as pltpu
from jax.experimental.pallas import tpu_sc as plsc

_B = 16384
_D = 64
_OUT = 128

_NC = 2
_NS = 16
_NW = _NC * _NS
_CHUNK = 128
_NROWS = _B // _CHUNK
_ROWS_PER_W = _NROWS // _NW


def _sc_gather(task_ids, table):
    idx2d = task_ids.reshape(_NROWS, _CHUNK).astype(jnp.int32)
    mesh = plsc.VectorSubcoreMesh(core_axis_name="c", subcore_axis_name="s")

    @functools.partial(
        pl.kernel,
        out_type=jax.ShapeDtypeStruct((_NROWS, _CHUNK, _D), jnp.float32),
        mesh=mesh,
        scratch_types=[
            pltpu.VMEM((_ROWS_PER_W, _CHUNK), jnp.int32),
            pltpu.VMEM((_ROWS_PER_W, _CHUNK, _D), jnp.float32),
            pltpu.SemaphoreType.DMA,
        ],
        compiler_params=pltpu.CompilerParams(use_tc_tiling_on_sc=False),
    )
    def gather_kernel(idx_hbm, table_hbm, out_hbm, idx_v, rows_v, sem):
        wid = lax.axis_index("s") * _NC + lax.axis_index("c")
        base = wid * _ROWS_PER_W
        pltpu.sync_copy(idx_hbm.at[pl.ds(base, _ROWS_PER_W)], idx_v)
        copies = [
            pltpu.async_copy(table_hbm.at[idx_v.at[j]], rows_v.at[j], sem)
            for j in range(_ROWS_PER_W)
        ]
        for c in copies:
            c.wait()
        pltpu.sync_copy(rows_v, out_hbm.at[pl.ds(base, _ROWS_PER_W)])

    return gather_kernel(idx2d, table).reshape(_B, _D)


_SQRT_HALF = 0.7071067811865476


def _proj_body(e_ref, w_ref, b_ref, o_ref):
    x = lax.dot_general(
        e_ref[...], w_ref[...],
        (((1,), (1,)), ((), ())),
        preferred_element_type=jnp.float32,
    )
    x = x + b_ref[...]
    o_ref[...] = x * 0.5 * (1.0 + lax.erf(x * _SQRT_HALF))


def _tc_proj(embed, W, b):
    blk = 2048
    return pl.pallas_call(
        _proj_body,
        grid=(_B // blk,),
        in_specs=[
            pl.BlockSpec((blk, _D), lambda i: (i, 0)),
            pl.BlockSpec((_OUT, _D), lambda i: (0, 0)),
            pl.BlockSpec((1, _OUT), lambda i: (0, 0)),
        ],
        out_specs=pl.BlockSpec((blk, _OUT), lambda i: (i, 0)),
        out_shape=jax.ShapeDtypeStruct((_B, _OUT), jnp.float32),
    )(embed, W, b.reshape(1, _OUT))


def kernel(task_ids, table, W, b):
    embed = _sc_gather(task_ids, table)
    proj = _tc_proj(embed, W, b)
    return (embed, proj)


# speedup vs baseline: 1.7074x; 1.7074x over previous
import functools

import jax
import jax.numpy as jnp
from jax import lax
from jax.experimental import pallas as pl
from jax.experimental.pallas import tpu as pltpu
from jax.experimental.pallas import tpu_sc as plsc

_B = 16384
_D = 64
_OUT = 128

_NC = 2
_NS = 16
_NW = _NC * _NS
_PW = _B // _NW
_UNROLL = 16


def _sc_gather(task_ids, table):
    idx2d = task_ids.reshape(_NW, _PW).astype(jnp.int32)
    mesh = plsc.VectorSubcoreMesh(core_axis_name="c", subcore_axis_name="s")

    @functools.partial(
        pl.kernel,
        out_type=jax.ShapeDtypeStruct((_NW, _PW, _D), jnp.float32),
        mesh=mesh,
        scratch_types=[
            pltpu.VMEM((1, _PW), jnp.int32),
            pltpu.VMEM((_PW, _D), jnp.float32),
            pltpu.SemaphoreType.DMA,
        ],
    )
    def gather_kernel(idx_hbm, table_hbm, out_hbm, idx_v, rows_v, sem):
        wid = lax.axis_index("s") * _NC + lax.axis_index("c")
        pltpu.sync_copy(idx_hbm.at[pl.ds(wid, 1)], idx_v)

        def step(j, _):
            vec = idx_v[0, pl.ds(j * _UNROLL, _UNROLL)]
            for u in range(_UNROLL):
                pltpu.async_copy(
                    table_hbm.at[pl.ds(vec[u], 1)],
                    rows_v.at[pl.ds(j * _UNROLL + u, 1)],
                    sem,
                )
            return ()

        lax.fori_loop(0, _PW // _UNROLL, step, ())
        pltpu.make_async_copy(table_hbm.at[pl.ds(0, _PW)], rows_v, sem).wait()
        pltpu.sync_copy(rows_v, out_hbm.at[wid])

    return gather_kernel(idx2d, table).reshape(_B, _D)


_SQRT_HALF = 0.7071067811865476


def _proj_body(e_ref, w_ref, b_ref, o_ref):
    x = lax.dot_general(
        e_ref[...], w_ref[...],
        (((1,), (1,)), ((), ())),
        preferred_element_type=jnp.float32,
    )
    x = x + b_ref[...]
    o_ref[...] = x * 0.5 * (1.0 + lax.erf(x * _SQRT_HALF))


def _tc_proj(embed, W, b):
    blk = 2048
    return pl.pallas_call(
        _proj_body,
        grid=(_B // blk,),
        in_specs=[
            pl.BlockSpec((blk, _D), lambda i: (i, 0)),
            pl.BlockSpec((_OUT, _D), lambda i: (0, 0)),
            pl.BlockSpec((1, _OUT), lambda i: (0, 0)),
        ],
        out_specs=pl.BlockSpec((blk, _OUT), lambda i: (i, 0)),
        out_shape=jax.ShapeDtypeStruct((_B, _OUT), jnp.float32),
    )(embed, W, b.reshape(1, _OUT))


def kernel(task_ids, table, W, b):
    embed = _sc_gather(task_ids, table)
    proj = _tc_proj(embed, W, b)
    return (embed, proj)


# speedup vs baseline: 1.8534x; 1.0855x over previous
import functools

import jax
import jax.numpy as jnp
from jax import lax
from jax.experimental import pallas as pl
from jax.experimental.pallas import tpu as pltpu
from jax.experimental.pallas import tpu_sc as plsc

_B = 16384
_D = 64
_OUT = 128
_V = 1000000
_LAST_BLK = _V // 128
_LAST_START = _LAST_BLK * 128

_NC = 2
_NS = 16
_NW = _NC * _NS
_PW = _B // _NW
_GRP = 16


def _sc_gather(task_ids, table_t):
    idx2d = task_ids.reshape(_NW, _PW).astype(jnp.int32)
    mesh = plsc.VectorSubcoreMesh(core_axis_name="c", subcore_axis_name="s")

    @functools.partial(
        pl.kernel,
        out_type=jax.ShapeDtypeStruct((_NW, _PW, _D), jnp.float32),
        mesh=mesh,
        scratch_types=[
            pltpu.VMEM((1, _PW), jnp.int32),
            pltpu.VMEM((2, _D, 128), jnp.float32),
            pltpu.VMEM((_PW, _D), jnp.float32),
            pltpu.SemaphoreType.DMA,
            pltpu.SemaphoreType.DMA,
        ],
        compiler_params=pltpu.CompilerParams(
            disable_bounds_checks=True, needs_layout_passes=False),
    )
    def gather_kernel(idx_hbm, table_hbm, out_hbm, idx_v, blk_v, rows_v, s0, s1):
        wid = lax.axis_index("s") * _NC + lax.axis_index("c")
        pltpu.sync_copy(idx_hbm.at[pl.ds(wid, 1)], idx_v)
        sems = (s0, s1)

        def issue(i, slot, sem):
            bstart = pl.multiple_of(lax.shift_right_logical(i, 7) * 128, 128)
            pltpu.async_copy(
                table_hbm.at[:, pl.ds(bstart, 128)], blk_v.at[slot], sem)

        def wait(slot, sem):
            pltpu.make_async_copy(
                table_hbm.at[:, pl.ds(0, 128)], blk_v.at[slot], sem).wait()

        def extract(i, j, slot):
            c = jnp.broadcast_to(lax.bitwise_and(i, 127), (16,))
            for k in range(_D // 16):
                dvec = lax.iota(jnp.int32, 16) + k * 16
                val = plsc.load_gather(blk_v.at[slot], [dvec, c])
                rows_v[j, pl.ds(k * 16, 16)] = val

        def group(g, _):
            vec = idx_v[0, pl.ds(g * _GRP, _GRP)]
            nxt = idx_v[0, pl.ds(jnp.minimum(g + 1, _PW // _GRP - 1) * _GRP, _GRP)]
            last_g = g == _PW // _GRP - 1
            for u in range(_GRP):
                if u + 1 < _GRP:
                    issue(vec[u + 1], (u + 1) % 2, sems[(u + 1) % 2])
                else:
                    @pl.when(jnp.logical_not(last_g))
                    def _():
                        issue(nxt[0], 0, sems[0])
                wait(u % 2, sems[u % 2])
                extract(vec[u], g * _GRP + u, u % 2)
            return ()

        first = idx_v[0, pl.ds(0, _GRP)]
        issue(first[0], 0, sems[0])
        lax.fori_loop(0, _PW // _GRP, group, ())
        pltpu.sync_copy(rows_v, out_hbm.at[wid])

    return gather_kernel(idx2d, table_t).reshape(_B, _D)


_SQRT_HALF = 0.7071067811865476


def _proj_body(e_ref, w_ref, b_ref, o_ref):
    x = lax.dot_general(
        e_ref[...], w_ref[...],
        (((1,), (1,)), ((), ())),
        preferred_element_type=jnp.float32,
    )
    x = x + b_ref[...]
    o_ref[...] = x * 0.5 * (1.0 + lax.erf(x * _SQRT_HALF))


def _tc_proj(embed, W, b):
    blk = 2048
    return pl.pallas_call(
        _proj_body,
        grid=(_B // blk,),
        in_specs=[
            pl.BlockSpec((blk, _D), lambda i: (i, 0)),
            pl.BlockSpec((_OUT, _D), lambda i: (0, 0)),
            pl.BlockSpec((1, _OUT), lambda i: (0, 0)),
        ],
        out_specs=pl.BlockSpec((blk, _OUT), lambda i: (i, 0)),
        out_shape=jax.ShapeDtypeStruct((_B, _OUT), jnp.float32),
    )(embed, W, b.reshape(1, _OUT))


def kernel(task_ids, table, W, b):
    embed = _sc_gather(task_ids, table.T)
    proj = _tc_proj(embed, W, b)
    return (embed, proj)


# speedup vs baseline: 2.4605x; 1.3276x over previous
import functools

import jax
import jax.numpy as jnp
from jax import lax
from jax.experimental import pallas as pl
from jax.experimental.pallas import tpu as pltpu
from jax.experimental.pallas import tpu_sc as plsc

_B = 16384
_D = 64
_OUT = 128
_V = 1000000
_LAST_BLK = _V // 128
_LAST_START = _LAST_BLK * 128

_NC = 2
_NS = 16
_NW = _NC * _NS
_PW = _B // _NW
_GRP = 16


def _sc_gather(task_ids, table_t):
    idx2d = task_ids.reshape(_NW, _PW).astype(jnp.int32)
    mesh = plsc.VectorSubcoreMesh(core_axis_name="c", subcore_axis_name="s")

    @functools.partial(
        pl.kernel,
        out_type=jax.ShapeDtypeStruct((_NW, _PW, _D), jnp.float32),
        mesh=mesh,
        scratch_types=[
            pltpu.VMEM((1, _PW), jnp.int32),
            pltpu.VMEM((4, _D, 128), jnp.float32),
            pltpu.VMEM((_PW, _D), jnp.float32),
            pltpu.SemaphoreType.DMA,
            pltpu.SemaphoreType.DMA,
            pltpu.SemaphoreType.DMA,
            pltpu.SemaphoreType.DMA,
        ],
        compiler_params=pltpu.CompilerParams(
            disable_bounds_checks=True, needs_layout_passes=False),
    )
    def gather_kernel(idx_hbm, table_hbm, out_hbm, idx_v, blk_v, rows_v,
                      s0, s1, s2, s3):
        wid = lax.axis_index("s") * _NC + lax.axis_index("c")
        pltpu.sync_copy(idx_hbm.at[pl.ds(wid, 1)], idx_v)
        sems = (s0, s1, s2, s3)

        def issue(i, slot, sem):
            bstart = pl.multiple_of(lax.shift_right_logical(i, 7) * 128, 128)
            pltpu.async_copy(
                table_hbm.at[:, pl.ds(bstart, 128)], blk_v.at[slot], sem)

        def wait(slot, sem):
            pltpu.make_async_copy(
                table_hbm.at[:, pl.ds(0, 128)], blk_v.at[slot], sem).wait()

        def extract(i, j, slot):
            c = jnp.broadcast_to(lax.bitwise_and(i, 127), (16,))
            for k in range(_D // 16):
                dvec = lax.iota(jnp.int32, 16) + k * 16
                val = plsc.load_gather(blk_v.at[slot], [dvec, c])
                rows_v[j, pl.ds(k * 16, 16)] = val

        _AHEAD = 3
        _NGRP = _PW // _GRP

        def group(g, _):
            vec = idx_v[0, pl.ds(g * _GRP, _GRP)]
            nxt = idx_v[0, pl.ds(jnp.minimum(g + 1, _NGRP - 1) * _GRP, _GRP)]
            not_last = g < _NGRP - 1
            for u in range(_GRP):
                a = u + _AHEAD
                slot = a % 4
                if a < _GRP:
                    issue(vec[a], slot, sems[slot])
                else:
                    @pl.when(not_last)
                    def _():
                        issue(nxt[a - _GRP], slot, sems[slot])
                wait(u % 4, sems[u % 4])
                extract(vec[u], g * _GRP + u, u % 4)
            return ()

        first = idx_v[0, pl.ds(0, _GRP)]
        for u in range(_AHEAD):
            issue(first[u], u % 4, sems[u % 4])
        lax.fori_loop(0, _NGRP, group, ())
        pltpu.sync_copy(rows_v, out_hbm.at[wid])

    return gather_kernel(idx2d, table_t).reshape(_B, _D)


_SQRT_HALF = 0.7071067811865476


def _proj_body(e_ref, w_ref, b_ref, o_ref):
    x = lax.dot_general(
        e_ref[...], w_ref[...],
        (((1,), (1,)), ((), ())),
        preferred_element_type=jnp.float32,
    )
    x = x + b_ref[...]
    o_ref[...] = x * 0.5 * (1.0 + lax.erf(x * _SQRT_HALF))


def _tc_proj(embed, W, b):
    blk = 2048
    return pl.pallas_call(
        _proj_body,
        grid=(_B // blk,),
        in_specs=[
            pl.BlockSpec((blk, _D), lambda i: (i, 0)),
            pl.BlockSpec((_OUT, _D), lambda i: (0, 0)),
            pl.BlockSpec((1, _OUT), lambda i: (0, 0)),
        ],
        out_specs=pl.BlockSpec((blk, _OUT), lambda i: (i, 0)),
        out_shape=jax.ShapeDtypeStruct((_B, _OUT), jnp.float32),
    )(embed, W, b.reshape(1, _OUT))


def kernel(task_ids, table, W, b):
    embed = _sc_gather(task_ids, table.T)
    proj = _tc_proj(embed, W, b)
    return (embed, proj)


# speedup vs baseline: 2.8895x; 1.1744x over previous
import functools

import jax
import jax.numpy as jnp
from jax import lax
from jax.experimental import pallas as pl
from jax.experimental.pallas import tpu as pltpu
from jax.experimental.pallas import tpu_sc as plsc

_B = 16384
_D = 64
_OUT = 128
_V = 1000000
_LAST_BLK = _V // 128
_LAST_START = _LAST_BLK * 128

_NC = 2
_NS = 16
_NW = _NC * _NS
_PW = _B // _NW
_GRP = 16


def _sc_gather(task_ids, table_t):
    idx2d = task_ids.reshape(_NW, _PW).astype(jnp.int32)
    mesh = plsc.VectorSubcoreMesh(core_axis_name="c", subcore_axis_name="s")

    @functools.partial(
        pl.kernel,
        out_type=jax.ShapeDtypeStruct((_NW, _PW, _D), jnp.float32),
        mesh=mesh,
        scratch_types=[
            pltpu.VMEM((1, _PW), jnp.int32),
            pltpu.VMEM((8, _D, 128), jnp.float32),
            pltpu.VMEM((128, _D), jnp.float32),
            [pltpu.SemaphoreType.DMA] * 8,
        ],
        compiler_params=pltpu.CompilerParams(
            disable_bounds_checks=True, needs_layout_passes=False),
    )
    def gather_kernel(idx_hbm, table_hbm, out_hbm, idx_v, blk_v, rows_v, sems):
        wid = lax.axis_index("s") * _NC + lax.axis_index("c")
        pltpu.sync_copy(idx_hbm.at[pl.ds(wid, 1)], idx_v)

        def issue(i, slot, sem):
            bstart = pl.multiple_of(lax.shift_right_logical(i, 7) * 128, 128)
            pltpu.async_copy(
                table_hbm.at[:, pl.ds(bstart, 128)], blk_v.at[slot], sem)

        def wait(slot, sem):
            pltpu.make_async_copy(
                table_hbm.at[:, pl.ds(0, 128)], blk_v.at[slot], sem).wait()

        def extract(i, j, slot):
            c = jnp.broadcast_to(lax.bitwise_and(i, 127), (16,))
            for k in range(_D // 16):
                dvec = lax.iota(jnp.int32, 16) + k * 16
                val = plsc.load_gather(blk_v.at[slot], [dvec, c])
                rows_v[j, pl.ds(k * 16, 16)] = val

        _AHEAD = 7
        _RING = 8
        _NGRP = _PW // _GRP
        _GPC = 128 // _GRP

        def group(g, _):
            vec = idx_v[0, pl.ds(g * _GRP, _GRP)]
            nxt = idx_v[0, pl.ds(jnp.minimum(g + 1, _NGRP - 1) * _GRP, _GRP)]
            not_last = g < _NGRP - 1
            jbase = (g % _GPC) * _GRP
            for u in range(_GRP):
                a = u + _AHEAD
                slot = a % _RING
                if a < _GRP:
                    issue(vec[a], slot, sems[slot])
                else:
                    @pl.when(not_last)
                    def _():
                        issue(nxt[a - _GRP], slot, sems[slot])
                wait(u % _RING, sems[u % _RING])
                extract(vec[u], jbase + u, u % _RING)

            @pl.when(g % _GPC == _GPC - 1)
            def _():
                pltpu.sync_copy(
                    rows_v, out_hbm.at[wid, pl.ds((g // _GPC) * 128, 128)])
            return ()

        first = idx_v[0, pl.ds(0, _GRP)]
        for u in range(_AHEAD):
            issue(first[u], u % _RING, sems[u % _RING])
        lax.fori_loop(0, _NGRP, group, ())

    return gather_kernel(idx2d, table_t).reshape(_B, _D)


_SQRT_HALF = 0.7071067811865476


def _proj_body(e_ref, w_ref, b_ref, o_ref):
    x = lax.dot_general(
        e_ref[...], w_ref[...],
        (((1,), (1,)), ((), ())),
        preferred_element_type=jnp.float32,
    )
    x = x + b_ref[...]
    o_ref[...] = x * 0.5 * (1.0 + lax.erf(x * _SQRT_HALF))


def _tc_proj(embed, W, b):
    blk = 2048
    return pl.pallas_call(
        _proj_body,
        grid=(_B // blk,),
        in_specs=[
            pl.BlockSpec((blk, _D), lambda i: (i, 0)),
            pl.BlockSpec((_OUT, _D), lambda i: (0, 0)),
            pl.BlockSpec((1, _OUT), lambda i: (0, 0)),
        ],
        out_specs=pl.BlockSpec((blk, _OUT), lambda i: (i, 0)),
        out_shape=jax.ShapeDtypeStruct((_B, _OUT), jnp.float32),
    )(embed, W, b.reshape(1, _OUT))


def kernel(task_ids, table, W, b):
    embed = _sc_gather(task_ids, table.T)
    proj = _tc_proj(embed, W, b)
    return (embed, proj)
